# baseline (speedup 1.0000x reference)
import jax
import jax.numpy as jnp
from jax import lax
from jax.experimental import pallas as pl
from jax.experimental.pallas import tpu as pltpu, tpu_sc as plsc

VOCAB = 1000000
DIM = 64
BATCH = 16384
FIELDS = 26

NC = 2
NS = 16
NW = NC * NS

N = BATCH * FIELDS
NPT = N // NW
BPT = BATCH // NW
SUB = 4 * FIELDS
CHUNK = 2 * SUB
BPC = 8
SPC = CHUNK // SUB
NCH = NPT // CHUNK
GROUPS = CHUNK // 2
TROW = 128

SHIFTS = (4, 0, 12, 8, 20, 16, 28, 24)


def _dequant_kernel(x_hbm, packed_hbm, absmax_hbm, cb_hbm, out_hbm,
                    idx_v, qidx_v, rows_v0, rows_v1, am_v0, am_v1,
                    out_v0, out_v1, cb_v, gsem0, gsem1, osem0, osem1):
    wid = lax.axis_index("s") * NC + lax.axis_index("c")

    pltpu.sync_copy(cb_hbm, cb_v)
    pltpu.sync_copy(x_hbm.at[pl.ds(wid * (NPT // SUB), NPT // SUB)], idx_v)

    iota = lax.iota(jnp.int32, 16)
    i3 = iota >> 3
    col = iota & 7
    dms = [(iota & 7) * 8 + m for m in range(8)]

    def qstage(k, _):
        flat = k * 16 + iota
        lr = (flat * 631) >> 16
        lc = flat - lr * SUB
        r = plsc.load_gather(idx_v, [lr, lc])
        plsc.store_scatter(qidx_v, [lr, lc], r >> 4)
        return _

    lax.fori_loop(0, NPT // 16, qstage, None)

    rows = (rows_v0, rows_v1)
    ams = (am_v0, am_v1)
    outs = (out_v0, out_v1)
    gsems = (gsem0, gsem1)
    osems = (osem0, osem1)

    def issue_gathers(ci, b):
        for s in range(SPC):
            pltpu.async_copy(
                packed_hbm.at[qidx_v.at[ci * SPC + s]],
                rows[b].at[pl.ds(s * SUB, SUB)], gsems[b])
            pltpu.async_copy(
                absmax_hbm.at[idx_v.at[ci * SPC + s]],
                ams[b].at[pl.ds(s * SUB, SUB)], gsems[b])

    def wait_gathers(b):
        pltpu.make_async_copy(
            packed_hbm.at[pl.ds(0, CHUNK)], rows[b], gsems[b]).wait()
        pltpu.make_async_copy(
            absmax_hbm.at[pl.ds(0, CHUNK)], ams[b], gsems[b]).wait()

    def wait_out(b):
        pltpu.make_async_copy(
            outs[b], out_hbm.at[pl.ds(0, BPC)], osems[b]).wait()

    issue_gathers(0, 0)
    issue_gathers(1, 1)

    def pair_body(t, _):
        for u in range(2):
            ci = 2 * t + u
            rows_b, am_b, out_b = rows[u], ams[u], outs[u]

            wait_gathers(u)

            @pl.when(ci >= 2)
            def _():
                wait_out(u)

            @plsc.parallel_loop(0, GROUPS, unroll=4)
            def _(g):
                rowv = 2 * g + i3
                lr = (rowv * 631) >> 16
                rv = plsc.load_gather(idx_v, [ci * SPC + lr,
                                              rowv - lr * SUB])
                w = plsc.load_gather(rows_b,
                                     [rowv, ((rv & 15) << 3) + col])
                scale = plsc.load_gather(am_b, [rowv])
                bv = (rowv * 2521) >> 16
                fv = rowv - bv * FIELDS
                for m in range(8):
                    q = (w >> SHIFTS[m]) & 15
                    plsc.store_scatter(out_b, [bv, fv, dms[m]],
                                       plsc.load_gather(cb_v, [q]) * scale)

            pltpu.async_copy(
                out_b,
                out_hbm.at[pl.ds(wid * BPT + ci * BPC, BPC)],
                osems[u])

            @pl.when(ci + 2 < NCH)
            def _():
                issue_gathers(ci + 2, u)
        return _

    lax.fori_loop(0, NCH // 2, pair_body, None)
    wait_out(0)
    wait_out(1)


def kernel(x, packed, absmax, codebook):
    packed32 = lax.bitcast_convert_type(
        packed.reshape(VOCAB, DIM // 8, 4), jnp.int32).reshape(
            VOCAB // 16, TROW)
    x2d = x.reshape(N // SUB, SUB)

    k = pl.kernel(
        _dequant_kernel,
        out_type=jax.ShapeDtypeStruct((BATCH, FIELDS, DIM), jnp.float32),
        mesh=plsc.VectorSubcoreMesh(
            core_axis_name="c", subcore_axis_name="s",
            num_cores=NC, num_subcores=NS),
        scratch_types=[
            pltpu.VMEM((NPT // SUB, SUB), jnp.int32),
            pltpu.VMEM((NPT // SUB, SUB), jnp.int32),
            pltpu.VMEM((CHUNK, TROW), jnp.int32),
            pltpu.VMEM((CHUNK, TROW), jnp.int32),
            pltpu.VMEM((CHUNK,), jnp.float32),
            pltpu.VMEM((CHUNK,), jnp.float32),
            pltpu.VMEM((BPC, FIELDS, DIM), jnp.float32),
            pltpu.VMEM((BPC, FIELDS, DIM), jnp.float32),
            pltpu.VMEM((16,), jnp.float32),
            pltpu.SemaphoreType.DMA,
            pltpu.SemaphoreType.DMA,
            pltpu.SemaphoreType.DMA,
            pltpu.SemaphoreType.DMA,
        ],
        compiler_params=pltpu.CompilerParams(
            needs_layout_passes=False, use_tc_tiling_on_sc=False),
    )
    return k(x2d, packed32, absmax, codebook)


# speedup vs baseline: 1.0229x; 1.0229x over previous
import jax
import jax.numpy as jnp
from jax import lax
from jax.experimental import pallas as pl
from jax.experimental.pallas import tpu as pltpu, tpu_sc as plsc

VOCAB = 1000000
DIM = 64
BATCH = 16384
FIELDS = 26

NC = 2
NS = 16
NW = NC * NS

N = BATCH * FIELDS
NPT = N // NW
BPT = BATCH // NW
SUB = 4 * FIELDS
CHUNK = 4 * SUB
BPC = 16
SPC = CHUNK // SUB
NCH = NPT // CHUNK
GROUPS = CHUNK // 2

SHIFTS = (4, 0, 12, 8, 20, 16, 28, 24)


def _dequant_kernel(x_hbm, packed_hbm, absmax_hbm, cb_hbm, out_hbm,
                    idx_v, rows_v0, rows_v1, am_v0, am_v1, out_v0, out_v1,
                    cb_v, gsem0, gsem1, osem0, osem1):
    wid = lax.axis_index("s") * NC + lax.axis_index("c")

    pltpu.sync_copy(cb_hbm, cb_v)
    pltpu.sync_copy(x_hbm.at[pl.ds(wid * (NPT // SUB), NPT // SUB)], idx_v)

    iota = lax.iota(jnp.int32, 16)
    i3 = iota >> 3
    col = iota & 7
    dms = [(iota & 7) * 8 + m for m in range(8)]

    rows = (rows_v0, rows_v1)
    ams = (am_v0, am_v1)
    outs = (out_v0, out_v1)
    gsems = (gsem0, gsem1)
    osems = (osem0, osem1)

    def issue_gathers(ci, b):
        for s in range(SPC):
            ir = idx_v.at[ci * SPC + s]
            pltpu.async_copy(
                packed_hbm.at[ir], rows[b].at[pl.ds(s * SUB, SUB)], gsems[b])
            pltpu.async_copy(
                absmax_hbm.at[ir], ams[b].at[pl.ds(s * SUB, SUB)], gsems[b])

    def wait_gathers(b):
        pltpu.make_async_copy(
            packed_hbm.at[pl.ds(0, CHUNK)], rows[b], gsems[b]).wait()
        pltpu.make_async_copy(
            absmax_hbm.at[pl.ds(0, CHUNK)], ams[b], gsems[b]).wait()

    def wait_out(b):
        pltpu.make_async_copy(
            outs[b], out_hbm.at[pl.ds(0, BPC)], osems[b]).wait()

    issue_gathers(0, 0)
    issue_gathers(1, 1)

    def pair_body(t, _):
        for u in range(2):
            ci = 2 * t + u
            rows_b, am_b, out_b = rows[u], ams[u], outs[u]

            wait_gathers(u)

            @pl.when(ci >= 2)
            def _():
                wait_out(u)

            @plsc.parallel_loop(0, GROUPS, unroll=4)
            def _(g):
                rowv = 2 * g + i3
                w = plsc.load_gather(rows_b, [rowv, col])
                scale = plsc.load_gather(am_b, [rowv])
                bv = (rowv * 2521) >> 16
                fv = rowv - bv * FIELDS
                for m in range(8):
                    q = (w >> SHIFTS[m]) & 15
                    plsc.store_scatter(out_b, [bv, fv, dms[m]],
                                       plsc.load_gather(cb_v, [q]) * scale)

            pltpu.async_copy(
                out_b,
                out_hbm.at[pl.ds(wid * BPT + ci * BPC, BPC)],
                osems[u])

            @pl.when(ci + 2 < NCH)
            def _():
                issue_gathers(ci + 2, u)
        return _

    lax.fori_loop(0, NCH // 2, pair_body, None)
    wait_out(0)
    wait_out(1)


def kernel(x, packed, absmax, codebook):
    packed32 = lax.bitcast_convert_type(
        packed.reshape(VOCAB, DIM // 8, 4), jnp.int32)
    x2d = x.reshape(N // SUB, SUB)

    k = pl.kernel(
        _dequant_kernel,
        out_type=jax.ShapeDtypeStruct((BATCH, FIELDS, DIM), jnp.float32),
        mesh=plsc.VectorSubcoreMesh(
            core_axis_name="c", subcore_axis_name="s",
            num_cores=NC, num_subcores=NS),
        scratch_types=[
            pltpu.VMEM((NPT // SUB, SUB), jnp.int32),
            pltpu.VMEM((CHUNK, DIM // 8), jnp.int32),
            pltpu.VMEM((CHUNK, DIM // 8), jnp.int32),
            pltpu.VMEM((CHUNK,), jnp.float32),
            pltpu.VMEM((CHUNK,), jnp.float32),
            pltpu.VMEM((BPC, FIELDS, DIM), jnp.float32),
            pltpu.VMEM((BPC, FIELDS, DIM), jnp.float32),
            pltpu.VMEM((16,), jnp.float32),
            pltpu.SemaphoreType.DMA,
            pltpu.SemaphoreType.DMA,
            pltpu.SemaphoreType.DMA,
            pltpu.SemaphoreType.DMA,
        ],
        compiler_params=pltpu.CompilerParams(
            needs_layout_passes=False, use_tc_tiling_on_sc=False),
    )
    return k(x2d, packed32, absmax, codebook)
